# Initial kernel scaffold; baseline (speedup 1.0000x reference)
import jax
import jax.numpy as jnp
from jax.experimental import pallas as pl


def kernel(x, edge_index, batch, Wl1, Wr1, b1, g1, be1, Wl2, Wr2, b2, g2, be2, Wl3, Wr3, b3, g3, be3, Wl4, Wr4, b4, g4, be4):
    raise NotImplementedError("write your pallas kernel here")


# speedup vs baseline: 1.0687x; 1.0687x over previous
import jax
import jax.numpy as jnp
from jax.experimental import pallas as pl

_N = 100000
_E = 1600000
_B = 256
_ROWS = 1000


def _dense_body(agg_ref, cnt_ref, h_ref, wl_ref, wr_ref, b_ref, g_ref, be_ref, out_ref):
    agg = agg_ref[...]
    cnt = cnt_ref[...]
    h = h_ref[...]
    mean = agg / jnp.maximum(cnt, 1.0)
    z = (jnp.dot(mean, wl_ref[...], preferred_element_type=jnp.float32)
         + jnp.dot(h, wr_ref[...], preferred_element_type=jnp.float32)
         + b_ref[...])
    m = jnp.mean(z, axis=-1, keepdims=True)
    v = jnp.mean((z - m) ** 2, axis=-1, keepdims=True)
    zn = (z - m) * jax.lax.rsqrt(v + 1e-5) * g_ref[...] + be_ref[...]
    out_ref[...] = 0.5 * zn * (1.0 + jax.lax.erf(zn * 0.7071067811865476))


def _dense(agg, cnt, h, wl, wr, b, g, be):
    n, d = h.shape
    grid = (n // _ROWS,)
    return pl.pallas_call(
        _dense_body,
        grid=grid,
        in_specs=[
            pl.BlockSpec((_ROWS, agg.shape[1]), lambda i: (i, 0)),
            pl.BlockSpec((_ROWS, 1), lambda i: (i, 0)),
            pl.BlockSpec((_ROWS, d), lambda i: (i, 0)),
            pl.BlockSpec((d, 128), lambda i: (0, 0)),
            pl.BlockSpec((d, 128), lambda i: (0, 0)),
            pl.BlockSpec((1, 128), lambda i: (0, 0)),
            pl.BlockSpec((1, 128), lambda i: (0, 0)),
            pl.BlockSpec((1, 128), lambda i: (0, 0)),
        ],
        out_specs=pl.BlockSpec((_ROWS, 128), lambda i: (i, 0)),
        out_shape=jax.ShapeDtypeStruct((n, 128), jnp.float32),
    )(agg, cnt, h, wl, wr, b, g, be)


def kernel(x, edge_index, batch,
           Wl1, Wr1, b1, g1, be1,
           Wl2, Wr2, b2, g2, be2,
           Wl3, Wr3, b3, g3, be3,
           Wl4, Wr4, b4, g4, be4):
    src = edge_index[0]
    dst = edge_index[1]
    cnt = jax.ops.segment_sum(jnp.ones((_E,), jnp.float32), dst,
                              num_segments=_N)[:, None]

    x16 = jnp.pad(x, ((0, 0), (0, 1)))
    wl1p = jnp.pad(Wl1, ((0, 1), (0, 0)))
    wr1p = jnp.pad(Wr1, ((0, 1), (0, 0)))
    agg = jax.ops.segment_sum(x16[src], dst, num_segments=_N)
    h = _dense(agg, cnt, x16, wl1p, wr1p, b1[None, :], g1[None, :], be1[None, :])

    for wl, wr, b, g, be in ((Wl2, Wr2, b2, g2, be2),
                             (Wl3, Wr3, b3, g3, be3),
                             (Wl4, Wr4, b4, g4, be4)):
        agg = jax.ops.segment_sum(h[src], dst, num_segments=_N)
        h = _dense(agg, cnt, h, wl, wr, b[None, :], g[None, :], be[None, :])

    pooled = jax.ops.segment_sum(h, batch, num_segments=_B)
    pcnt = jax.ops.segment_sum(jnp.ones((_N,), jnp.float32), batch,
                               num_segments=_B)
    return pooled / jnp.clip(pcnt, 1.0, None)[:, None]


# speedup vs baseline: 3.4777x; 3.2540x over previous
import functools

import jax
import jax.numpy as jnp
from jax import lax
from jax.experimental import pallas as pl
from jax.experimental.pallas import tpu as pltpu
from jax.experimental.pallas import tpu_sc as plsc

_N = 100000
_E = 1600000
_B = 256

_NT = 100096
_ACC = _NT + 8
_EPAD = 1605632
_ER = _EPAD // 128
_RPT = _NT // 16
_ROWS = 1000

_mesh = plsc.VectorSubcoreMesh(core_axis_name="c", subcore_axis_name="s")


def _zero_fill(zero_v):
    def zrow(i, _):
        zero_v[i] = jnp.zeros((16,), jnp.float32)
        return 0
    lax.fori_loop(0, zero_v.shape[0], zrow, 0)


def _sweep_edges(src_ref, dst_ref, table, acc, src_v, dst_v, rows_v, gsem,
                 row_base, n_chunks):
    def chunk(k, _):
        r0 = row_base + k * 8
        pltpu.sync_copy(src_ref.at[pl.ds(r0, 8)], src_v)
        pltpu.sync_copy(dst_ref.at[pl.ds(r0, 8)], dst_v)
        cps = [pltpu.async_copy(table.at[src_v.at[j]],
                                rows_v.at[pl.ds(j * 128, 128)], gsem)
               for j in range(8)]
        for cp in cps:
            cp.wait()
        for j in range(8):
            pltpu.sync_copy(rows_v.at[pl.ds(j * 128, 128)],
                            acc.at[dst_v.at[j]], add=True)
        return 0
    lax.fori_loop(0, n_chunks, chunk, 0)


def _zero_slice(acc, zero_v, base):
    for j in range(_RPT // 1024):
        pltpu.sync_copy(zero_v, acc.at[pl.ds(base + j * 1024, 1024)])
    rem = _RPT % 1024
    if rem:
        pltpu.sync_copy(zero_v.at[pl.ds(0, rem)],
                        acc.at[pl.ds(base + _RPT - rem, rem)])


def _agg_body(src_ref, dst_ref, t0, t1, t2, t3, t4, t5, t6, t7,
              o0, o1, o2, o3, o4, o5, o6, o7,
              acc, src_v, dst_v, rows_v, gsem):
    c = lax.axis_index("c")
    s = lax.axis_index("s")
    tables = (t0, t1, t2, t3, t4, t5, t6, t7)
    outs = (o0, o1, o2, o3, o4, o5, o6, o7)
    base = s * _RPT
    for g in range(8):
        @pl.when(c == g // 4)
        def _(g=g):
            _zero_fill(rows_v)
            _zero_slice(acc, rows_v, base)
            plsc.subcore_barrier()
            _sweep_edges(src_ref, dst_ref, tables[g], acc,
                         src_v, dst_v, rows_v, gsem,
                         s * (_ER // 16), _ER // (16 * 8))
            plsc.subcore_barrier()
            pltpu.sync_copy(acc.at[pl.ds(base, _RPT)],
                            outs[g].at[pl.ds(base, _RPT)])
            plsc.subcore_barrier()


def _agg1_body(src_ref, dst_ref, x_ref, o0, o1,
               acc, src_v, dst_v, rows_v, gsem):
    c = lax.axis_index("c")
    s = lax.axis_index("s")
    base = s * _RPT
    _zero_fill(rows_v)
    _zero_slice(acc, rows_v, base)
    plsc.subcore_barrier()
    _sweep_edges(src_ref, dst_ref, x_ref, acc, src_v, dst_v, rows_v, gsem,
                 c * (_ER // 2) + s * (_ER // 32), _ER // (32 * 8))
    plsc.subcore_barrier()

    @pl.when(c == 0)
    def _():
        pltpu.sync_copy(acc.at[pl.ds(base, _RPT)], o0.at[pl.ds(base, _RPT)])

    @pl.when(c == 1)
    def _():
        pltpu.sync_copy(acc.at[pl.ds(base, _RPT)], o1.at[pl.ds(base, _RPT)])


_sc_scratch = [
    pltpu.VMEM_SHARED((_ACC, 16), jnp.float32),
    pltpu.VMEM((8, 128), jnp.int32),
    pltpu.VMEM((8, 128), jnp.int32),
    pltpu.VMEM((1024, 16), jnp.float32),
    pltpu.SemaphoreType.DMA,
]

_sc_params = pltpu.CompilerParams(use_tc_tiling_on_sc=False)

_agg = pl.kernel(
    _agg_body,
    out_type=tuple(jax.ShapeDtypeStruct((_NT, 16), jnp.float32)
                   for _ in range(8)),
    mesh=_mesh,
    scratch_types=_sc_scratch,
    compiler_params=_sc_params,
)

_agg1 = pl.kernel(
    _agg1_body,
    out_type=tuple(jax.ShapeDtypeStruct((_NT, 16), jnp.float32)
                   for _ in range(2)),
    mesh=_mesh,
    scratch_types=_sc_scratch,
    compiler_params=_sc_params,
)


def _norm_act(z, g_ref, be_ref):
    m = jnp.mean(z, axis=-1, keepdims=True)
    v = jnp.mean((z - m) ** 2, axis=-1, keepdims=True)
    zn = (z - m) * lax.rsqrt(v + 1e-5) * g_ref[...] + be_ref[...]
    return 0.5 * zn * (1.0 + lax.erf(zn * 0.7071067811865476))


def _dense1_body(p0_ref, p1_ref, x_ref, wl_ref, wr_ref, b_ref, g_ref, be_ref,
                 out_ref):
    agg = p0_ref[...] + p1_ref[...]
    cnt = agg[:, 15:16]
    mean = agg / jnp.maximum(cnt, 1.0)
    z = (jnp.dot(mean, wl_ref[...], preferred_element_type=jnp.float32)
         + jnp.dot(x_ref[...], wr_ref[...], preferred_element_type=jnp.float32)
         + b_ref[...])
    out_ref[...] = _norm_act(z, g_ref, be_ref)


def _dense_body(agg_ref, cnt_ref, h_ref, wl_ref, wr_ref, b_ref, g_ref, be_ref,
                out_ref):
    mean = agg_ref[...] / jnp.maximum(cnt_ref[...], 1.0)
    z = (jnp.dot(mean, wl_ref[...], preferred_element_type=jnp.float32)
         + jnp.dot(h_ref[...], wr_ref[...], preferred_element_type=jnp.float32)
         + b_ref[...])
    out_ref[...] = _norm_act(z, g_ref, be_ref)


def _dense4_body(agg_ref, cnt_ref, h_ref, wl_ref, wr_ref, b_ref, g_ref,
                 be_ref, batch_ref, pool_ref, pcnt_ref):
    mean = agg_ref[...] / jnp.maximum(cnt_ref[...], 1.0)
    z = (jnp.dot(mean, wl_ref[...], preferred_element_type=jnp.float32)
         + jnp.dot(h_ref[...], wr_ref[...], preferred_element_type=jnp.float32)
         + b_ref[...])
    h4 = _norm_act(z, g_ref, be_ref)
    oh = (batch_ref[...] == lax.broadcasted_iota(jnp.int32, (1, _B), 1)
          ).astype(jnp.float32)
    pool_blk = lax.dot_general(oh, h4, (((0,), (0,)), ((), ())),
                               preferred_element_type=jnp.float32)
    cnt_blk = jnp.sum(oh, axis=0, keepdims=True)

    @pl.when(pl.program_id(0) == 0)
    def _():
        pool_ref[...] = pool_blk
        pcnt_ref[...] = cnt_blk

    @pl.when(pl.program_id(0) > 0)
    def _():
        pool_ref[...] += pool_blk
        pcnt_ref[...] += cnt_blk


def _row_spec(d):
    return pl.BlockSpec((_ROWS, d), lambda i: (i, 0))


def _w_spec(d):
    return pl.BlockSpec((d, 128), lambda i: (0, 0))


_VEC_SPECS = [pl.BlockSpec((1, 128), lambda i: (0, 0))] * 3


def _dense1(p0, p1, x16, wl, wr, b, g, be):
    return pl.pallas_call(
        _dense1_body,
        grid=(_N // _ROWS,),
        in_specs=[_row_spec(16)] * 3 + [_w_spec(16)] * 2 + _VEC_SPECS,
        out_specs=_row_spec(128),
        out_shape=jax.ShapeDtypeStruct((_N, 128), jnp.float32),
    )(p0, p1, x16, wl, wr, b, g, be)


def _dense(agg, cnt, h, wl, wr, b, g, be):
    return pl.pallas_call(
        _dense_body,
        grid=(_N // _ROWS,),
        in_specs=[_row_spec(128), _row_spec(1), _row_spec(128)]
        + [_w_spec(128)] * 2 + _VEC_SPECS,
        out_specs=_row_spec(128),
        out_shape=jax.ShapeDtypeStruct((_N, 128), jnp.float32),
    )(agg, cnt, h, wl, wr, b, g, be)


def _dense4(agg, cnt, h, wl, wr, b, g, be, batch2d):
    return pl.pallas_call(
        _dense4_body,
        grid=(_N // _ROWS,),
        in_specs=[_row_spec(128), _row_spec(1), _row_spec(128)]
        + [_w_spec(128)] * 2 + _VEC_SPECS
        + [pl.BlockSpec((_ROWS, 1), lambda i: (i, 0))],
        out_specs=[pl.BlockSpec((_B, 128), lambda i: (0, 0)),
                   pl.BlockSpec((1, _B), lambda i: (0, 0))],
        out_shape=[jax.ShapeDtypeStruct((_B, 128), jnp.float32),
                   jax.ShapeDtypeStruct((1, _B), jnp.float32)],
    )(agg, cnt, h, wl, wr, b, g, be, batch2d)


def kernel(x, edge_index, batch,
           Wl1, Wr1, b1, g1, be1,
           Wl2, Wr2, b2, g2, be2,
           Wl3, Wr3, b3, g3, be3,
           Wl4, Wr4, b4, g4, be4):
    src = edge_index[0]
    dst = edge_index[1]
    npad = _EPAD - _E
    pad_ids = jnp.arange(npad, dtype=jnp.int32)
    src2d = jnp.concatenate([src, pad_ids % 128]).reshape(_ER, 128)
    dst2d = jnp.concatenate([dst, _NT + (pad_ids % 8)]).reshape(_ER, 128)

    x16 = jnp.concatenate([x, jnp.ones((_N, 1), jnp.float32)], axis=1)
    wl1p = jnp.pad(Wl1, ((0, 1), (0, 0)))
    wr1p = jnp.pad(Wr1, ((0, 1), (0, 0)))

    p0, p1 = _agg1(src2d, dst2d, x16)
    cnt = (p0[:_N, 15] + p1[:_N, 15])[:, None]
    h = _dense1(p0[:_N], p1[:_N], x16, wl1p, wr1p,
                b1[None, :], g1[None, :], be1[None, :])

    for wl, wr, b, g, be, last in ((Wl2, Wr2, b2, g2, be2, False),
                                   (Wl3, Wr3, b3, g3, be3, False),
                                   (Wl4, Wr4, b4, g4, be4, True)):
        parts = _agg(src2d, dst2d,
                     *[h[:, 16 * j:16 * (j + 1)] for j in range(8)])
        agg = jnp.concatenate([o[:_N] for o in parts], axis=1)
        if not last:
            h = _dense(agg, cnt, h, wl, wr,
                       b[None, :], g[None, :], be[None, :])
        else:
            pooled, pcnt = _dense4(agg, cnt, h, wl, wr,
                                   b[None, :], g[None, :], be[None, :],
                                   batch[:, None])
    return pooled / jnp.clip(pcnt[0], 1.0, None)[:, None]
